# baseline (speedup 1.0000x reference)
import functools

import jax
import jax.numpy as jnp
from jax import lax
from jax.experimental import pallas as pl
from jax.experimental.pallas import tpu as pltpu
from jax.experimental.pallas import tpu_sc as plsc

N_POINTS = 1048576
IN_DIM = 3
N_FEATS = 8
HASHMAP_SIZE = 524288
HASH_MASK = HASHMAP_SIZE - 1
RES = 512.0
PRIME1 = -1640531535
PRIME2 = 805459861

NC = 2
NS = 16
NW = NC * NS
NP = N_POINTS // NW
C = 512
G = NP // C
NIDX = N_FEATS * C
IROWS = NIDX // 128
XW = 4


def _body(x_hbm, table_hbm, out_hbm, xb, ib, wb, rb, ob, gsem, osem):
    wid = lax.axis_index("s") * NC + lax.axis_index("c")
    base_pt = wid * NP
    i16 = lax.iota(jnp.int32, 16)
    fcol = [jnp.full((16,), f, jnp.int32) for f in range(N_FEATS)]

    def stage(g, par):
        pbase = base_pt + g * C
        xbuf, idxbuf, wbuf = xb[par], ib[par], wb[par]
        pltpu.sync_copy(x_hbm.at[pl.ds(pbase * XW, C * XW)], xbuf)

        @pl.loop(0, C // 16)
        def _p1(t):
            xoff = (t >> 3) * (128 * XW) + (t & 7) * 16
            h0, h1, xf, om = [], [], [], []
            for d in range(IN_DIM):
                xs = xbuf[pl.ds(xoff + d * 128, 16)] * RES
                xi = xs.astype(jnp.int32)
                frac = xs - xi.astype(jnp.float32)
                xf.append(frac)
                om.append(1.0 - frac)
                if d == 0:
                    h0.append(xi)
                    h1.append(xi + 1)
                else:
                    p = PRIME1 if d == 1 else PRIME2
                    hp = xi * p
                    h0.append(hp)
                    h1.append(hp + p)
            for j in range(1 << IN_DIM):
                hid = None
                w = None
                for d in range(IN_DIM):
                    bit = (j >> d) & 1
                    hd = h1[d] if bit else h0[d]
                    wd = xf[d] if bit else om[d]
                    hid = hd if hid is None else hid ^ hd
                    w = wd if w is None else w * wd
                hid = hid & HASH_MASK
                flat = j * C + t * 16
                idxbuf[pl.ds(flat, 16)] = hid
                wbuf[pl.ds(flat, 16)] = w

        pltpu.async_copy(table_hbm.at[idxbuf], rb[par], gsem[par])

    def combine(g, par):
        pbase = base_pt + g * C
        wbuf, rowsbuf, outbuf = wb[par], rb[par], ob[par]
        pltpu.make_async_copy(
            table_hbm.at[pl.ds(0, NIDX)], rowsbuf, gsem[par]
        ).wait()

        @pl.when(g >= 2)
        def _():
            pltpu.make_async_copy(
                x_hbm.at[pl.ds(0, C * N_FEATS)], outbuf, osem[par]
            ).wait()

        @pl.loop(0, C // 16)
        def _p3(t):
            ooff = (t >> 3) * (128 * N_FEATS) + (t & 7) * 16
            accs = [None] * N_FEATS
            for j in range(1 << IN_DIM):
                flat = j * C + t * 16
                wv = wbuf[pl.ds(flat, 16)]
                rid = flat + i16
                for f in range(N_FEATS):
                    rv = plsc.load_gather(rowsbuf, [rid, fcol[f]])
                    term = rv * wv
                    accs[f] = term if accs[f] is None else accs[f] + term
            for f in range(N_FEATS):
                outbuf[pl.ds(ooff + f * 128, 16)] = accs[f]

        pltpu.async_copy(
            outbuf, out_hbm.at[pl.ds(pbase * N_FEATS, C * N_FEATS)], osem[par]
        )

    stage(0, 0)

    @pl.loop(0, G // 2)
    def _gg(gg):
        g0 = 2 * gg
        stage(g0 + 1, 1)
        combine(g0, 0)

        @pl.when(g0 + 2 < G)
        def _():
            stage(g0 + 2, 0)

        combine(g0 + 1, 1)

    pltpu.make_async_copy(x_hbm.at[pl.ds(0, C * N_FEATS)], ob[0], osem[0]).wait()
    pltpu.make_async_copy(x_hbm.at[pl.ds(0, C * N_FEATS)], ob[1], osem[1]).wait()


@functools.partial(
    pl.kernel,
    out_type=jax.ShapeDtypeStruct((N_POINTS * N_FEATS,), jnp.float32),
    mesh=plsc.VectorSubcoreMesh(
        core_axis_name="c", subcore_axis_name="s", num_cores=NC, num_subcores=NS
    ),
    compiler_params=pltpu.CompilerParams(
        needs_layout_passes=False, use_tc_tiling_on_sc=False
    ),
    scratch_types=[
        [pltpu.VMEM((C * XW,), jnp.float32)] * 2,
        [pltpu.VMEM((NIDX,), jnp.int32)] * 2,
        [pltpu.VMEM((NIDX,), jnp.float32)] * 2,
        [pltpu.VMEM((NIDX, N_FEATS), jnp.float32)] * 2,
        [pltpu.VMEM((C * N_FEATS,), jnp.float32)] * 2,
        [pltpu.SemaphoreType.DMA] * 2,
        [pltpu.SemaphoreType.DMA] * 2,
    ],
)
def _hash_grid(x_hbm, table_hbm, out_hbm, xb, ib, wb, rb, ob, gsem, osem):
    _body(x_hbm, table_hbm, out_hbm, xb, ib, wb, rb, ob, gsem, osem)


def kernel(x, table):
    xp = jnp.pad(x, ((0, 0), (0, XW - IN_DIM)))
    x_flat = xp.reshape(N_POINTS // 128, 128, XW).transpose(0, 2, 1).reshape(-1)
    out_flat = _hash_grid(x_flat, table)
    out = (
        out_flat.reshape(N_POINTS // 128, N_FEATS, 128)
        .transpose(0, 2, 1)
        .reshape(N_POINTS, N_FEATS)
    )
    return out


# speedup vs baseline: 1.1276x; 1.1276x over previous
import functools

import jax
import jax.numpy as jnp
from jax import lax
from jax.experimental import pallas as pl
from jax.experimental.pallas import tpu as pltpu
from jax.experimental.pallas import tpu_sc as plsc

N_POINTS = 1048576
IN_DIM = 3
N_FEATS = 8
HASHMAP_SIZE = 524288
HASH_MASK = HASHMAP_SIZE - 1
RES = 512.0
PRIME1 = -1640531535
PRIME2 = 805459861

NC = 2
NS = 16
NW = NC * NS
NP = N_POINTS // NW
C = 512
G = NP // C
NIDX = N_FEATS * C
XW = 4
NBLK = HASHMAP_SIZE // 128
BPT = NBLK // NS
BSTEP = 4


def _body(x_hbm, tn_hbm, out_hbm, tbl_hbm, xb, ib, wb, rb, ob, nb, tb,
          gsem, osem):
    cid = lax.axis_index("c")
    sid = lax.axis_index("s")
    wid = sid * NC + cid
    base_pt = wid * NP
    i16 = lax.iota(jnp.int32, 16)
    fcol = [jnp.full((16,), f, jnp.int32) for f in range(N_FEATS)]
    tbase = cid * HASHMAP_SIZE

    @pl.loop(0, BPT // BSTEP)
    def _conv(k):
        blk0 = sid * BPT + k * BSTEP
        pltpu.sync_copy(tn_hbm.at[pl.ds(blk0 * 1024, BSTEP * 1024)], nb)
        for bb in range(BSTEP):
            for lg in range(8):
                lane = bb * 128 + lg * 16 + i16
                for f in range(N_FEATS):
                    v = nb[pl.ds(bb * 1024 + f * 128 + lg * 16, 16)]
                    plsc.store_scatter(tb, [lane, fcol[f]], v)
        pltpu.sync_copy(tb, tbl_hbm.at[pl.ds(tbase + blk0 * 128, BSTEP * 128)])

    plsc.subcore_barrier()

    def stage(g, par):
        pbase = base_pt + g * C
        xbuf, idxbuf, wbuf = xb[par], ib[par], wb[par]
        pltpu.sync_copy(x_hbm.at[pl.ds(pbase * XW, C * XW)], xbuf)

        @pl.loop(0, C // 16)
        def _p1(t):
            xoff = (t >> 3) * (128 * XW) + (t & 7) * 16
            h0, h1, xf, om = [], [], [], []
            for d in range(IN_DIM):
                xs = xbuf[pl.ds(xoff + d * 128, 16)] * RES
                xi = xs.astype(jnp.int32)
                frac = xs - xi.astype(jnp.float32)
                xf.append(frac)
                om.append(1.0 - frac)
                if d == 0:
                    h0.append(xi)
                    h1.append(xi + 1)
                else:
                    p = PRIME1 if d == 1 else PRIME2
                    hp = xi * p
                    h0.append(hp)
                    h1.append(hp + p)
            for j in range(1 << IN_DIM):
                hid = None
                w = None
                for d in range(IN_DIM):
                    bit = (j >> d) & 1
                    hd = h1[d] if bit else h0[d]
                    wd = xf[d] if bit else om[d]
                    hid = hd if hid is None else hid ^ hd
                    w = wd if w is None else w * wd
                hid = (hid & HASH_MASK) + tbase
                flat = j * C + t * 16
                idxbuf[pl.ds(flat, 16)] = hid
                wbuf[pl.ds(flat, 16)] = w

        pltpu.async_copy(tbl_hbm.at[idxbuf], rb[par], gsem[par])

    def combine(g, par):
        pbase = base_pt + g * C
        wbuf, rowsbuf, outbuf = wb[par], rb[par], ob[par]
        pltpu.make_async_copy(
            tbl_hbm.at[pl.ds(0, NIDX)], rowsbuf, gsem[par]
        ).wait()

        @pl.when(g >= 2)
        def _():
            pltpu.make_async_copy(
                x_hbm.at[pl.ds(0, C * N_FEATS)], outbuf, osem[par]
            ).wait()

        @pl.loop(0, C // 16)
        def _p3(t):
            ooff = (t >> 3) * (128 * N_FEATS) + (t & 7) * 16
            accs = [None] * N_FEATS
            for j in range(1 << IN_DIM):
                flat = j * C + t * 16
                wv = wbuf[pl.ds(flat, 16)]
                rid = flat + i16
                for f in range(N_FEATS):
                    rv = plsc.load_gather(rowsbuf, [rid, fcol[f]])
                    term = rv * wv
                    accs[f] = term if accs[f] is None else accs[f] + term
            for f in range(N_FEATS):
                outbuf[pl.ds(ooff + f * 128, 16)] = accs[f]

        pltpu.async_copy(
            outbuf, out_hbm.at[pl.ds(pbase * N_FEATS, C * N_FEATS)], osem[par]
        )

    stage(0, 0)

    @pl.loop(0, G // 2)
    def _gg(gg):
        g0 = 2 * gg
        stage(g0 + 1, 1)
        combine(g0, 0)

        @pl.when(g0 + 2 < G)
        def _():
            stage(g0 + 2, 0)

        combine(g0 + 1, 1)

    pltpu.make_async_copy(x_hbm.at[pl.ds(0, C * N_FEATS)], ob[0], osem[0]).wait()
    pltpu.make_async_copy(x_hbm.at[pl.ds(0, C * N_FEATS)], ob[1], osem[1]).wait()


@functools.partial(
    pl.kernel,
    out_type=(
        jax.ShapeDtypeStruct((N_POINTS * N_FEATS,), jnp.float32),
        jax.ShapeDtypeStruct((NC * HASHMAP_SIZE, N_FEATS), jnp.float32),
    ),
    mesh=plsc.VectorSubcoreMesh(
        core_axis_name="c", subcore_axis_name="s", num_cores=NC, num_subcores=NS
    ),
    compiler_params=pltpu.CompilerParams(
        needs_layout_passes=False, use_tc_tiling_on_sc=False
    ),
    scratch_types=[
        [pltpu.VMEM((C * XW,), jnp.float32)] * 2,
        [pltpu.VMEM((NIDX,), jnp.int32)] * 2,
        [pltpu.VMEM((NIDX,), jnp.float32)] * 2,
        [pltpu.VMEM((NIDX, N_FEATS), jnp.float32)] * 2,
        [pltpu.VMEM((C * N_FEATS,), jnp.float32)] * 2,
        pltpu.VMEM((BSTEP * 1024,), jnp.float32),
        pltpu.VMEM((BSTEP * 128, N_FEATS), jnp.float32),
        [pltpu.SemaphoreType.DMA] * 2,
        [pltpu.SemaphoreType.DMA] * 2,
    ],
)
def _hash_grid(x_hbm, tn_hbm, out_hbm, tbl_hbm, xb, ib, wb, rb, ob, nb, tb,
               gsem, osem):
    _body(x_hbm, tn_hbm, out_hbm, tbl_hbm, xb, ib, wb, rb, ob, nb, tb,
          gsem, osem)


def kernel(x, table):
    xp = jnp.pad(x, ((0, 0), (0, XW - IN_DIM)))
    x_flat = xp.reshape(N_POINTS // 128, 128, XW).transpose(0, 2, 1).reshape(-1)
    t_nat = (
        table.reshape(HASHMAP_SIZE // 128, 128, N_FEATS)
        .transpose(0, 2, 1)
        .reshape(-1)
    )
    out_flat, _ = _hash_grid(x_flat, t_nat)
    out = (
        out_flat.reshape(N_POINTS // 128, N_FEATS, 128)
        .transpose(0, 2, 1)
        .reshape(N_POINTS, N_FEATS)
    )
    return out


# speedup vs baseline: 1.3113x; 1.1629x over previous
import functools

import jax
import jax.numpy as jnp
from jax import lax
from jax.experimental import pallas as pl
from jax.experimental.pallas import tpu as pltpu
from jax.experimental.pallas import tpu_sc as plsc

N_POINTS = 1048576
IN_DIM = 3
N_FEATS = 8
HASHMAP_SIZE = 524288
HASH_MASK = HASHMAP_SIZE - 1
RES = 512.0
PRIME1 = -1640531535
PRIME2 = 805459861

NC = 2
NS = 16
NW = NC * NS
NP = N_POINTS // NW
C = 512
G = NP // C
NIDX = N_FEATS * C
XW = 4
NBLK = HASHMAP_SIZE // 128
BPT = NBLK // NS
BSTEP = 4


def _body(x_hbm, tn_hbm, out_hbm, tbl_hbm, xb, ib, wb, rb, ob, nb, tb,
          gsem, osem, csem_i, csem_o):
    cid = lax.axis_index("c")
    sid = lax.axis_index("s")
    wid = sid * NC + cid
    base_pt = wid * NP
    i16 = lax.iota(jnp.int32, 16)
    fcol = [jnp.full((16,), f, jnp.int32) for f in range(N_FEATS)]
    tbase = cid * HASHMAP_SIZE

    NSTEP = BPT // BSTEP

    def conv_step(k, par):
        @pl.when(k + 1 < NSTEP)
        def _():
            nblk = sid * BPT + (k + 1) * BSTEP
            pltpu.async_copy(
                tn_hbm.at[pl.ds(nblk * 1024, BSTEP * 1024)],
                nb[1 - par],
                csem_i[1 - par],
            )

        pltpu.make_async_copy(
            tn_hbm.at[pl.ds(0, BSTEP * 1024)], nb[par], csem_i[par]
        ).wait()

        @pl.when(k >= 2)
        def _():
            pltpu.make_async_copy(
                tbl_hbm.at[pl.ds(0, BSTEP * 128)], tb[par], csem_o[par]
            ).wait()

        blk0 = sid * BPT + k * BSTEP
        for bb in range(BSTEP):
            for lg in range(8):
                lane = bb * 128 + lg * 16 + i16
                for f in range(N_FEATS):
                    v = nb[par][pl.ds(bb * 1024 + f * 128 + lg * 16, 16)]
                    plsc.store_scatter(tb[par], [lane, fcol[f]], v)
        pltpu.async_copy(
            tb[par],
            tbl_hbm.at[pl.ds(tbase + blk0 * 128, BSTEP * 128)],
            csem_o[par],
        )

    pltpu.async_copy(
        tn_hbm.at[pl.ds(sid * BPT * 1024, BSTEP * 1024)], nb[0], csem_i[0]
    )

    @pl.loop(0, NSTEP // 2)
    def _conv(kk):
        conv_step(2 * kk, 0)
        conv_step(2 * kk + 1, 1)

    for par in range(2):
        pltpu.make_async_copy(
            tbl_hbm.at[pl.ds(0, BSTEP * 128)], tb[par], csem_o[par]
        ).wait()

    plsc.subcore_barrier()

    def stage(g, par):
        pbase = base_pt + g * C
        xbuf, idxbuf, wbuf = xb[par], ib[par], wb[par]
        pltpu.sync_copy(x_hbm.at[pl.ds(pbase * XW, C * XW)], xbuf)

        @pl.loop(0, C // 16)
        def _p1(t):
            xoff = (t >> 3) * (128 * XW) + (t & 7) * 16
            h0, h1, xf, om = [], [], [], []
            for d in range(IN_DIM):
                xs = xbuf[pl.ds(xoff + d * 128, 16)] * RES
                xi = xs.astype(jnp.int32)
                frac = xs - xi.astype(jnp.float32)
                xf.append(frac)
                om.append(1.0 - frac)
                if d == 0:
                    h0.append(xi)
                    h1.append(xi + 1)
                else:
                    p = PRIME1 if d == 1 else PRIME2
                    hp = xi * p
                    h0.append(hp)
                    h1.append(hp + p)
            for j in range(1 << IN_DIM):
                hid = None
                w = None
                for d in range(IN_DIM):
                    bit = (j >> d) & 1
                    hd = h1[d] if bit else h0[d]
                    wd = xf[d] if bit else om[d]
                    hid = hd if hid is None else hid ^ hd
                    w = wd if w is None else w * wd
                hid = (hid & HASH_MASK) + tbase
                flat = j * C + t * 16
                idxbuf[pl.ds(flat, 16)] = hid
                wbuf[pl.ds(flat, 16)] = w

        pltpu.async_copy(tbl_hbm.at[idxbuf], rb[par], gsem[par])

    def combine(g, par):
        pbase = base_pt + g * C
        wbuf, rowsbuf, outbuf = wb[par], rb[par], ob[par]
        pltpu.make_async_copy(
            tbl_hbm.at[pl.ds(0, NIDX)], rowsbuf, gsem[par]
        ).wait()

        @pl.when(g >= 2)
        def _():
            pltpu.make_async_copy(
                x_hbm.at[pl.ds(0, C * N_FEATS)], outbuf, osem[par]
            ).wait()

        @pl.loop(0, C // 16)
        def _p3(t):
            ooff = (t >> 3) * (128 * N_FEATS) + (t & 7) * 16
            accs = [None] * N_FEATS
            for j in range(1 << IN_DIM):
                flat = j * C + t * 16
                wv = wbuf[pl.ds(flat, 16)]
                rid = flat + i16
                for f in range(N_FEATS):
                    rv = plsc.load_gather(rowsbuf, [rid, fcol[f]])
                    term = rv * wv
                    accs[f] = term if accs[f] is None else accs[f] + term
            for f in range(N_FEATS):
                outbuf[pl.ds(ooff + f * 128, 16)] = accs[f]

        pltpu.async_copy(
            outbuf, out_hbm.at[pl.ds(pbase * N_FEATS, C * N_FEATS)], osem[par]
        )

    stage(0, 0)

    @pl.loop(0, G // 2)
    def _gg(gg):
        g0 = 2 * gg
        stage(g0 + 1, 1)
        combine(g0, 0)

        @pl.when(g0 + 2 < G)
        def _():
            stage(g0 + 2, 0)

        combine(g0 + 1, 1)

    pltpu.make_async_copy(x_hbm.at[pl.ds(0, C * N_FEATS)], ob[0], osem[0]).wait()
    pltpu.make_async_copy(x_hbm.at[pl.ds(0, C * N_FEATS)], ob[1], osem[1]).wait()


@functools.partial(
    pl.kernel,
    out_type=(
        jax.ShapeDtypeStruct((N_POINTS * N_FEATS,), jnp.float32),
        jax.ShapeDtypeStruct((NC * HASHMAP_SIZE, N_FEATS), jnp.float32),
    ),
    mesh=plsc.VectorSubcoreMesh(
        core_axis_name="c", subcore_axis_name="s", num_cores=NC, num_subcores=NS
    ),
    compiler_params=pltpu.CompilerParams(
        needs_layout_passes=False, use_tc_tiling_on_sc=False
    ),
    scratch_types=[
        [pltpu.VMEM((C * XW,), jnp.float32)] * 2,
        [pltpu.VMEM((NIDX,), jnp.int32)] * 2,
        [pltpu.VMEM((NIDX,), jnp.float32)] * 2,
        [pltpu.VMEM((NIDX, N_FEATS), jnp.float32)] * 2,
        [pltpu.VMEM((C * N_FEATS,), jnp.float32)] * 2,
        [pltpu.VMEM((BSTEP * 1024,), jnp.float32)] * 2,
        [pltpu.VMEM((BSTEP * 128, N_FEATS), jnp.float32)] * 2,
        [pltpu.SemaphoreType.DMA] * 2,
        [pltpu.SemaphoreType.DMA] * 2,
        [pltpu.SemaphoreType.DMA] * 2,
        [pltpu.SemaphoreType.DMA] * 2,
    ],
)
def _hash_grid(x_hbm, tn_hbm, out_hbm, tbl_hbm, xb, ib, wb, rb, ob, nb, tb,
               gsem, osem, csem_i, csem_o):
    _body(x_hbm, tn_hbm, out_hbm, tbl_hbm, xb, ib, wb, rb, ob, nb, tb,
          gsem, osem, csem_i, csem_o)


def kernel(x, table):
    xp = jnp.pad(x, ((0, 0), (0, XW - IN_DIM)))
    x_flat = xp.reshape(N_POINTS // 128, 128, XW).transpose(0, 2, 1).reshape(-1)
    t_nat = (
        table.reshape(HASHMAP_SIZE // 128, 128, N_FEATS)
        .transpose(0, 2, 1)
        .reshape(-1)
    )
    out_flat, _ = _hash_grid(x_flat, t_nat)
    out = (
        out_flat.reshape(N_POINTS // 128, N_FEATS, 128)
        .transpose(0, 2, 1)
        .reshape(N_POINTS, N_FEATS)
    )
    return out


# speedup vs baseline: 1.3671x; 1.0425x over previous
import functools

import jax
import jax.numpy as jnp
from jax import lax
from jax.experimental import pallas as pl
from jax.experimental.pallas import tpu as pltpu
from jax.experimental.pallas import tpu_sc as plsc

N_POINTS = 1048576
IN_DIM = 3
N_FEATS = 8
HASHMAP_SIZE = 524288
HASH_MASK = HASHMAP_SIZE - 1
RES = 512.0
PRIME1 = -1640531535
PRIME2 = 805459861

NC = 2
NS = 16
NW = NC * NS
NP = N_POINTS // NW
C = 512
G = NP // C
NIDX = N_FEATS * C
XW = 4
NBLK = HASHMAP_SIZE // 128
BPT = NBLK // NS
BSTEP = 4


def _body(x_hbm, tn_hbm, out_hbm, tbl_hbm, xb, ib, wb, rb, ob, nb, tb,
          gsem, osem, csem_i, csem_o, xsem):
    cid = lax.axis_index("c")
    sid = lax.axis_index("s")
    wid = sid * NC + cid
    base_pt = wid * NP
    i16 = lax.iota(jnp.int32, 16)
    fcol = [jnp.full((16,), f, jnp.int32) for f in range(N_FEATS)]
    tbase = cid * HASHMAP_SIZE

    NSTEP = BPT // BSTEP

    def conv_step(k, par):
        @pl.when(k + 1 < NSTEP)
        def _():
            nblk = sid * BPT + (k + 1) * BSTEP
            pltpu.async_copy(
                tn_hbm.at[pl.ds(nblk * 1024, BSTEP * 1024)],
                nb[1 - par],
                csem_i[1 - par],
            )

        pltpu.make_async_copy(
            tn_hbm.at[pl.ds(0, BSTEP * 1024)], nb[par], csem_i[par]
        ).wait()

        @pl.when(k >= 2)
        def _():
            pltpu.make_async_copy(
                tbl_hbm.at[pl.ds(0, BSTEP * 128)], tb[par], csem_o[par]
            ).wait()

        blk0 = sid * BPT + k * BSTEP
        for bb in range(BSTEP):
            for lg in range(8):
                lane = bb * 128 + lg * 16 + i16
                for f in range(N_FEATS):
                    v = nb[par][pl.ds(bb * 1024 + f * 128 + lg * 16, 16)]
                    plsc.store_scatter(tb[par], [lane, fcol[f]], v)
        pltpu.async_copy(
            tb[par],
            tbl_hbm.at[pl.ds(tbase + blk0 * 128, BSTEP * 128)],
            csem_o[par],
        )

    pltpu.async_copy(
        tn_hbm.at[pl.ds(sid * BPT * 1024, BSTEP * 1024)], nb[0], csem_i[0]
    )

    @pl.loop(0, NSTEP // 2)
    def _conv(kk):
        conv_step(2 * kk, 0)
        conv_step(2 * kk + 1, 1)

    for par in range(2):
        pltpu.make_async_copy(
            tbl_hbm.at[pl.ds(0, BSTEP * 128)], tb[par], csem_o[par]
        ).wait()

    plsc.subcore_barrier()

    def stage(g, par):
        pbase = base_pt + g * C
        xbuf, idxbuf, wbuf = xb[par], ib[par], wb[par]
        pltpu.sync_copy(x_hbm.at[pl.ds(pbase * XW, C * XW)], xbuf)

        @pl.loop(0, C // 16)
        def _p1(t):
            xoff = (t >> 3) * (128 * XW) + (t & 7) * 16
            h0, h1, xf, om = [], [], [], []
            for d in range(IN_DIM):
                xs = xbuf[pl.ds(xoff + d * 128, 16)] * RES
                xi = xs.astype(jnp.int32)
                frac = xs - xi.astype(jnp.float32)
                xf.append(frac)
                om.append(1.0 - frac)
                if d == 0:
                    h0.append(xi)
                    h1.append(xi + 1)
                else:
                    p = PRIME1 if d == 1 else PRIME2
                    hp = xi * p
                    h0.append(hp)
                    h1.append(hp + p)
            for j in range(1 << IN_DIM):
                hid = None
                w = None
                for d in range(IN_DIM):
                    bit = (j >> d) & 1
                    hd = h1[d] if bit else h0[d]
                    wd = xf[d] if bit else om[d]
                    hid = hd if hid is None else hid ^ hd
                    w = wd if w is None else w * wd
                hid = (hid & HASH_MASK) + tbase
                flat = j * C + t * 16
                idxbuf[pl.ds(flat, 16)] = hid
                wbuf[pl.ds(flat, 16)] = w

        pltpu.async_copy(tbl_hbm.at[idxbuf], rb[par], gsem[par])

    def combine(g, par):
        pbase = base_pt + g * C
        wbuf, rowsbuf, outbuf = wb[par], rb[par], ob[par]
        pltpu.make_async_copy(
            tbl_hbm.at[pl.ds(0, NIDX)], rowsbuf, gsem[par]
        ).wait()

        @pl.when(g >= 2)
        def _():
            pltpu.make_async_copy(
                x_hbm.at[pl.ds(0, C * N_FEATS)], outbuf, osem[par]
            ).wait()

        @pl.loop(0, C // 16)
        def _p3(t):
            ooff = (t >> 3) * (128 * N_FEATS) + (t & 7) * 16
            accs = [None] * N_FEATS
            for j in range(1 << IN_DIM):
                flat = j * C + t * 16
                wv = wbuf[pl.ds(flat, 16)]
                rid = flat + i16
                for f in range(N_FEATS):
                    rv = plsc.load_gather(rowsbuf, [rid, fcol[f]])
                    term = rv * wv
                    accs[f] = term if accs[f] is None else accs[f] + term
            for f in range(N_FEATS):
                outbuf[pl.ds(ooff + f * 128, 16)] = accs[f]

        pltpu.async_copy(
            outbuf, out_hbm.at[pl.ds(pbase * N_FEATS, C * N_FEATS)], osem[par]
        )

    def fused(g, par):
        pbase = base_pt + g * C
        pltpu.async_copy(
            x_hbm.at[pl.ds((pbase + 2 * C) * XW, C * XW)], xb[par], xsem[par]
        )
        wbuf, rowsbuf, outbuf = wb[par], rb[par], ob[par]
        idxbuf, xbuf = ib[par], xb[par]
        pltpu.make_async_copy(
            tbl_hbm.at[pl.ds(0, NIDX)], rowsbuf, gsem[par]
        ).wait()

        @pl.when(g >= 2)
        def _():
            pltpu.make_async_copy(
                x_hbm.at[pl.ds(0, C * N_FEATS)], outbuf, osem[par]
            ).wait()

        pltpu.make_async_copy(
            x_hbm.at[pl.ds(0, C * XW)], xbuf, xsem[par]
        ).wait()

        @pl.loop(0, C // 16)
        def _pf(t):
            ooff = (t >> 3) * (128 * N_FEATS) + (t & 7) * 16
            accs = [None] * N_FEATS
            for j in range(1 << IN_DIM):
                flat = j * C + t * 16
                wv = wbuf[pl.ds(flat, 16)]
                rid = flat + i16
                for f in range(N_FEATS):
                    rv = plsc.load_gather(rowsbuf, [rid, fcol[f]])
                    term = rv * wv
                    accs[f] = term if accs[f] is None else accs[f] + term
            for f in range(N_FEATS):
                outbuf[pl.ds(ooff + f * 128, 16)] = accs[f]
            xoff = (t >> 3) * (128 * XW) + (t & 7) * 16
            h0, h1, xf, om = [], [], [], []
            for d in range(IN_DIM):
                xs = xbuf[pl.ds(xoff + d * 128, 16)] * RES
                xi = xs.astype(jnp.int32)
                frac = xs - xi.astype(jnp.float32)
                xf.append(frac)
                om.append(1.0 - frac)
                if d == 0:
                    h0.append(xi)
                    h1.append(xi + 1)
                else:
                    p = PRIME1 if d == 1 else PRIME2
                    hp = xi * p
                    h0.append(hp)
                    h1.append(hp + p)
            for j in range(1 << IN_DIM):
                hid = None
                w = None
                for d in range(IN_DIM):
                    bit = (j >> d) & 1
                    hd = h1[d] if bit else h0[d]
                    wd = xf[d] if bit else om[d]
                    hid = hd if hid is None else hid ^ hd
                    w = wd if w is None else w * wd
                hid = (hid & HASH_MASK) + tbase
                flat = j * C + t * 16
                idxbuf[pl.ds(flat, 16)] = hid
                wbuf[pl.ds(flat, 16)] = w

        pltpu.async_copy(
            outbuf, out_hbm.at[pl.ds(pbase * N_FEATS, C * N_FEATS)], osem[par]
        )
        pltpu.async_copy(tbl_hbm.at[idxbuf], rb[par], gsem[par])

    stage(0, 0)
    stage(1, 1)

    @pl.loop(0, (G - 2) // 2)
    def _gg(gg):
        g0 = 2 * gg
        fused(g0, 0)
        fused(g0 + 1, 1)

    combine(G - 2, 0)
    combine(G - 1, 1)

    pltpu.make_async_copy(x_hbm.at[pl.ds(0, C * N_FEATS)], ob[0], osem[0]).wait()
    pltpu.make_async_copy(x_hbm.at[pl.ds(0, C * N_FEATS)], ob[1], osem[1]).wait()


@functools.partial(
    pl.kernel,
    out_type=(
        jax.ShapeDtypeStruct((N_POINTS * N_FEATS,), jnp.float32),
        jax.ShapeDtypeStruct((NC * HASHMAP_SIZE, N_FEATS), jnp.float32),
    ),
    mesh=plsc.VectorSubcoreMesh(
        core_axis_name="c", subcore_axis_name="s", num_cores=NC, num_subcores=NS
    ),
    compiler_params=pltpu.CompilerParams(
        needs_layout_passes=False, use_tc_tiling_on_sc=False
    ),
    scratch_types=[
        [pltpu.VMEM((C * XW,), jnp.float32)] * 2,
        [pltpu.VMEM((NIDX,), jnp.int32)] * 2,
        [pltpu.VMEM((NIDX,), jnp.float32)] * 2,
        [pltpu.VMEM((NIDX, N_FEATS), jnp.float32)] * 2,
        [pltpu.VMEM((C * N_FEATS,), jnp.float32)] * 2,
        [pltpu.VMEM((BSTEP * 1024,), jnp.float32)] * 2,
        [pltpu.VMEM((BSTEP * 128, N_FEATS), jnp.float32)] * 2,
        [pltpu.SemaphoreType.DMA] * 2,
        [pltpu.SemaphoreType.DMA] * 2,
        [pltpu.SemaphoreType.DMA] * 2,
        [pltpu.SemaphoreType.DMA] * 2,
        [pltpu.SemaphoreType.DMA] * 2,
    ],
)
def _hash_grid(x_hbm, tn_hbm, out_hbm, tbl_hbm, xb, ib, wb, rb, ob, nb, tb,
               gsem, osem, csem_i, csem_o, xsem):
    _body(x_hbm, tn_hbm, out_hbm, tbl_hbm, xb, ib, wb, rb, ob, nb, tb,
          gsem, osem, csem_i, csem_o, xsem)


def kernel(x, table):
    xp = jnp.pad(x, ((0, 0), (0, XW - IN_DIM)))
    x_flat = xp.reshape(N_POINTS // 128, 128, XW).transpose(0, 2, 1).reshape(-1)
    t_nat = (
        table.reshape(HASHMAP_SIZE // 128, 128, N_FEATS)
        .transpose(0, 2, 1)
        .reshape(-1)
    )
    out_flat, _ = _hash_grid(x_flat, t_nat)
    out = (
        out_flat.reshape(N_POINTS // 128, N_FEATS, 128)
        .transpose(0, 2, 1)
        .reshape(N_POINTS, N_FEATS)
    )
    return out
